# baseline (speedup 1.0000x reference)
import functools

import jax
import jax.numpy as jnp
from jax import lax
from jax.experimental import pallas as pl
from jax.experimental.pallas import tpu as pltpu
from jax.experimental.pallas import tpu_sc as plsc

N, C, H, W = 4, 96, 192, 192
OH, OW = 2 * H, 2 * W
NC = N * C
NUM_CORES = 1
NUM_SUBCORES = 16
NW = NUM_CORES * NUM_SUBCORES

CH = 32
CHUNKS_PER_PLANE = H // CH
IN_CHUNK = CH * W
OUT_CHUNK = 4 * IN_CHUNK
TOTAL_CHUNKS = NC * CHUNKS_PER_PLANE
CHUNKS_PER_WORKER = TOTAL_CHUNKS // NW
LANES = 16

_mesh = plsc.VectorSubcoreMesh(
    core_axis_name="c", subcore_axis_name="s",
    num_cores=NUM_CORES, num_subcores=NUM_SUBCORES)


@functools.partial(
    pl.kernel,
    out_type=jax.ShapeDtypeStruct((NC * OH * OW,), jnp.float32),
    mesh=_mesh,
    scratch_types=[
        pltpu.VMEM((IN_CHUNK,), jnp.float32),
        pltpu.VMEM((IN_CHUNK,), jnp.float32),
        pltpu.VMEM((IN_CHUNK,), jnp.int32),
        pltpu.VMEM((IN_CHUNK,), jnp.int32),
        pltpu.VMEM((OUT_CHUNK,), jnp.float32),
        pltpu.VMEM((OUT_CHUNK,), jnp.float32),
        pltpu.SemaphoreType.DMA,
        pltpu.SemaphoreType.DMA,
        pltpu.SemaphoreType.DMA,
        pltpu.SemaphoreType.DMA,
        pltpu.SemaphoreType.DMA,
        pltpu.SemaphoreType.DMA,
    ],
    compiler_params=pltpu.CompilerParams(needs_layout_passes=False),
)
def _sc_unpool(x_hbm, idx_hbm, out_hbm,
               xv0, xv1, iv0, iv1, ov0, ov1,
               sx0, sx1, si0, si1, so0, so1):
    wid = lax.axis_index("s") * NUM_CORES + lax.axis_index("c")
    xv = (xv0, xv1)
    iv = (iv0, iv1)
    ov = (ov0, ov1)
    sx = (sx0, sx1)
    si = (si0, si1)
    so = (so0, so1)

    def in_slice(g):
        gchunk = wid * CHUNKS_PER_WORKER + g
        plane = gchunk // CHUNKS_PER_PLANE
        rc = gchunk % CHUNKS_PER_PLANE
        return plane * (H * W) + rc * IN_CHUNK, rc

    def start_in(g, b):
        off, _ = in_slice(g)
        pltpu.async_copy(x_hbm.at[pl.ds(off, IN_CHUNK)], xv[b], sx[b])
        pltpu.async_copy(idx_hbm.at[pl.ds(off, IN_CHUNK)], iv[b], si[b])

    start_in(0, 0)

    def pair_body(p, _):
        for b in range(2):
            g = p * 2 + b
            nb = (b + 1) % 2

            @pl.when(g + 1 < CHUNKS_PER_WORKER)
            def _():
                start_in(g + 1, nb)

            pltpu.make_async_copy(
                x_hbm.at[pl.ds(0, IN_CHUNK)], xv[b], sx[b]).wait()
            pltpu.make_async_copy(
                idx_hbm.at[pl.ds(0, IN_CHUNK)], iv[b], si[b]).wait()

            @pl.when(g >= 2)
            def _():
                pltpu.make_async_copy(
                    ov[b], out_hbm.at[pl.ds(0, OUT_CHUNK)], so[b]).wait()

            gchunk = wid * CHUNKS_PER_WORKER + g
            plane = gchunk // CHUNKS_PER_PLANE
            rc = gchunk % CHUNKS_PER_PLANE
            base = rc * OUT_CHUNK
            out_off = plane * (OH * OW) + rc * OUT_CHUNK

            zeros = jnp.zeros((LANES,), jnp.float32)
            ovb = ov[b]
            ivb = iv[b]
            xvb = xv[b]

            @plsc.parallel_loop(0, OUT_CHUNK // LANES, unroll=8)
            def zero_body(k):
                ovb[pl.ds(k * LANES, LANES)] = zeros

            @plsc.parallel_loop(0, IN_CHUNK // LANES, unroll=8)
            def scatter_body(k):
                sl = pl.ds(k * LANES, LANES)
                local = ivb[sl] - base
                plsc.store_scatter(ovb, [local], xvb[sl])

            pltpu.async_copy(ovb, out_hbm.at[pl.ds(out_off, OUT_CHUNK)],
                             so[b])
        return 0

    lax.fori_loop(0, CHUNKS_PER_WORKER // 2, pair_body, 0)

    for b in range(2):
        pltpu.make_async_copy(
            ov[b], out_hbm.at[pl.ds(0, OUT_CHUNK)], so[b]).wait()


def kernel(x, indices):
    xf = x.reshape(NC * H * W)
    idxf = indices.reshape(NC * H * W)
    out = _sc_unpool(xf, idxf)
    return out.reshape(N, C, OH, OW)


# speedup vs baseline: 1.1325x; 1.1325x over previous
import functools

import jax
import jax.numpy as jnp
from jax import lax
from jax.experimental import pallas as pl
from jax.experimental.pallas import tpu as pltpu
from jax.experimental.pallas import tpu_sc as plsc

N, C, H, W = 4, 96, 192, 192
OH, OW = 2 * H, 2 * W
NC = N * C
NUM_CORES = 2
NUM_SUBCORES = 16
NW = NUM_CORES * NUM_SUBCORES

CH = 48
CHUNKS_PER_PLANE = H // CH
IN_CHUNK = CH * W
OUT_CHUNK = 4 * IN_CHUNK
TOTAL_CHUNKS = NC * CHUNKS_PER_PLANE
CHUNKS_PER_WORKER = TOTAL_CHUNKS // NW
LANES = 16

_mesh = plsc.VectorSubcoreMesh(
    core_axis_name="c", subcore_axis_name="s",
    num_cores=NUM_CORES, num_subcores=NUM_SUBCORES)


@functools.partial(
    pl.kernel,
    out_type=jax.ShapeDtypeStruct((NC * OH * OW,), jnp.float32),
    mesh=_mesh,
    scratch_types=[
        pltpu.VMEM((IN_CHUNK,), jnp.float32),
        pltpu.VMEM((IN_CHUNK,), jnp.float32),
        pltpu.VMEM((IN_CHUNK,), jnp.int32),
        pltpu.VMEM((IN_CHUNK,), jnp.int32),
        pltpu.VMEM((OUT_CHUNK,), jnp.float32),
        pltpu.VMEM((OUT_CHUNK,), jnp.float32),
        pltpu.SemaphoreType.DMA,
        pltpu.SemaphoreType.DMA,
        pltpu.SemaphoreType.DMA,
        pltpu.SemaphoreType.DMA,
        pltpu.SemaphoreType.DMA,
        pltpu.SemaphoreType.DMA,
    ],
    compiler_params=pltpu.CompilerParams(needs_layout_passes=False),
)
def _sc_unpool(x_hbm, idx_hbm, out_hbm,
               xv0, xv1, iv0, iv1, ov0, ov1,
               sx0, sx1, si0, si1, so0, so1):
    wid = lax.axis_index("s") * NUM_CORES + lax.axis_index("c")
    xv = (xv0, xv1)
    iv = (iv0, iv1)
    ov = (ov0, ov1)
    sx = (sx0, sx1)
    si = (si0, si1)
    so = (so0, so1)

    def in_slice(g):
        gchunk = wid * CHUNKS_PER_WORKER + g
        plane = gchunk // CHUNKS_PER_PLANE
        rc = gchunk % CHUNKS_PER_PLANE
        return plane * (H * W) + rc * IN_CHUNK, rc

    def start_in(g, b):
        off, _ = in_slice(g)
        pltpu.async_copy(x_hbm.at[pl.ds(off, IN_CHUNK)], xv[b], sx[b])
        pltpu.async_copy(idx_hbm.at[pl.ds(off, IN_CHUNK)], iv[b], si[b])

    start_in(0, 0)

    def pair_body(p, _):
        for b in range(2):
            g = p * 2 + b
            nb = (b + 1) % 2

            @pl.when(g + 1 < CHUNKS_PER_WORKER)
            def _():
                start_in(g + 1, nb)

            pltpu.make_async_copy(
                x_hbm.at[pl.ds(0, IN_CHUNK)], xv[b], sx[b]).wait()
            pltpu.make_async_copy(
                idx_hbm.at[pl.ds(0, IN_CHUNK)], iv[b], si[b]).wait()

            @pl.when(g >= 2)
            def _():
                pltpu.make_async_copy(
                    ov[b], out_hbm.at[pl.ds(0, OUT_CHUNK)], so[b]).wait()

            gchunk = wid * CHUNKS_PER_WORKER + g
            plane = gchunk // CHUNKS_PER_PLANE
            rc = gchunk % CHUNKS_PER_PLANE
            base = rc * OUT_CHUNK
            out_off = plane * (OH * OW) + rc * OUT_CHUNK

            zeros = jnp.zeros((LANES,), jnp.float32)
            ovb = ov[b]
            ivb = iv[b]
            xvb = xv[b]

            @plsc.parallel_loop(0, OUT_CHUNK // LANES, unroll=8)
            def zero_body(k):
                ovb[pl.ds(k * LANES, LANES)] = zeros

            @plsc.parallel_loop(0, IN_CHUNK // LANES, unroll=8)
            def scatter_body(k):
                sl = pl.ds(k * LANES, LANES)
                local = ivb[sl] - base
                plsc.store_scatter(ovb, [local], xvb[sl])

            pltpu.async_copy(ovb, out_hbm.at[pl.ds(out_off, OUT_CHUNK)],
                             so[b])
        return 0

    lax.fori_loop(0, CHUNKS_PER_WORKER // 2, pair_body, 0)

    for b in range(2):
        pltpu.make_async_copy(
            ov[b], out_hbm.at[pl.ds(0, OUT_CHUNK)], so[b]).wait()


def kernel(x, indices):
    xf = x.reshape(NC * H * W)
    idxf = indices.reshape(NC * H * W)
    out = _sc_unpool(xf, idxf)
    return out.reshape(N, C, OH, OW)
